# baseline (speedup 1.0000x reference)
import jax
import jax.numpy as jnp
from jax import lax
from jax.experimental import pallas as pl
from jax.experimental.pallas import tpu as pltpu
from jax.experimental.pallas import tpu_sc as plsc

B = 16384
NN = 13
NCAT = 26
CARD = 100000
D = 32
NTOK = 1 + NN + NCAT

NC = 2
NS = 16
NW = NC * NS
BW = B // NW
SLAB = 256
NSLAB = BW // SLAB
NSTG = 6


def _bc(x):
    return jnp.broadcast_to(x, (16,))


def _sc_body(numT, catT, w_hbm, bias_hbm, cls_hbm, tabT, out,
             craw, nraw, stg, w_v, bias_v, cls_v,
             gs0, gs1, gs2, gs3, wsem):
    gsems = [gs0, gs1, gs2, gs3]
    wid = lax.axis_index("s") * NC + lax.axis_index("c")
    base = pl.multiple_of(wid * BW, BW)

    pltpu.sync_copy(w_hbm, w_v)
    pltpu.sync_copy(bias_hbm, bias_v)
    pltpu.sync_copy(cls_hbm, cls_v)

    def slab(s, carry):
        b0 = pl.multiple_of(base + s * SLAB, SLAB)
        pltpu.sync_copy(catT.at[:, pl.ds(b0, SLAB)], craw)
        pltpu.sync_copy(numT.at[:, pl.ds(b0, SLAB)], nraw)

        wh = [None] * NSTG

        def prefire(i):
            slot = 2 + (i % 4)
            if wh[slot] is not None:
                wh[slot].wait()
                wh[slot] = None
            idx = craw.at[i]
            sem = gsems[i % 4]

            def dfire(d, c):
                pltpu.async_copy(tabT.at[i, d].at[idx], stg.at[slot, d], sem)
                return c
            lax.fori_loop(0, D, dfire, 0)

        def emit(t, slot):
            wh[slot] = pltpu.async_copy(
                stg.at[slot], out.at[pl.ds(t * D, D), pl.ds(b0, SLAB)], wsem)

        for i in range(3):
            prefire(i)

        def fill_cls(st):
            def dloop(d, c):
                cv = plsc.load_gather(cls_v, [_bc(d)])
                for k in range(SLAB // 16):
                    st[d, pl.ds(16 * k, 16)] = cv
                return c
            lax.fori_loop(0, D, dloop, 0)
        fill_cls(stg.at[0])
        emit(0, 0)

        for j in range(NN):
            slot = (j + 1) % 2
            if wh[slot] is not None:
                wh[slot].wait()

            def fill_num(st, j=j):
                def dloop(d, c):
                    bw = plsc.load_gather(w_v, [_bc(j), _bc(d)])
                    bb = plsc.load_gather(bias_v, [_bc(j), _bc(d)])
                    for k in range(SLAB // 16):
                        st[d, pl.ds(16 * k, 16)] = nraw[j, pl.ds(16 * k, 16)] * bw + bb
                    return c
                lax.fori_loop(0, D, dloop, 0)
            fill_num(stg.at[slot])
            emit(1 + j, slot)

        for i in range(NCAT):
            slot = 2 + (i % 4)
            pltpu.make_async_copy(
                tabT.at[0].at[:, pl.ds(0, SLAB)], stg.at[slot],
                gsems[i % 4]).wait()
            emit(1 + NN + i, slot)
            if i + 3 < NCAT:
                prefire(i + 3)

        for h in wh:
            if h is not None:
                h.wait()
        return carry
    lax.fori_loop(0, NSLAB, slab, 0)


def kernel(numeric, categorical, num_weight, num_bias, cat_tables, cls_token):
    numT = numeric.T
    catT = categorical.T
    tabT = cat_tables.transpose(0, 2, 1)
    cls = cls_token.reshape(D)
    mesh = plsc.VectorSubcoreMesh(core_axis_name="c", subcore_axis_name="s")
    fn = pl.kernel(
        _sc_body,
        out_type=jax.ShapeDtypeStruct((NTOK * D, B), jnp.float32),
        mesh=mesh,
        scratch_types=[
            pltpu.VMEM((NCAT, SLAB), jnp.int32),
            pltpu.VMEM((NN, SLAB), jnp.float32),
            pltpu.VMEM((NSTG, D, SLAB), jnp.float32),
            pltpu.VMEM((NN, D), jnp.float32),
            pltpu.VMEM((NN, D), jnp.float32),
            pltpu.VMEM((D,), jnp.float32),
            pltpu.SemaphoreType.DMA,
            pltpu.SemaphoreType.DMA,
            pltpu.SemaphoreType.DMA,
            pltpu.SemaphoreType.DMA,
            pltpu.SemaphoreType.DMA,
        ],
        compiler_params=pltpu.CompilerParams(use_tc_tiling_on_sc=False,
                                             needs_layout_passes=False),
    )
    out = fn(numT, catT, num_weight, num_bias, cls, tabT)
    return out.reshape(NTOK, D, B).transpose(2, 0, 1)


# speedup vs baseline: 1.0080x; 1.0080x over previous
import jax
import jax.numpy as jnp
from jax import lax
from jax.experimental import pallas as pl
from jax.experimental.pallas import tpu as pltpu
from jax.experimental.pallas import tpu_sc as plsc

B = 16384
NN = 13
NCAT = 26
CARD = 100000
D = 32
NTOK = 1 + NN + NCAT

NC = 2
NS = 16
NW = NC * NS
BW = B // NW
SLAB = 128
NSLAB = BW // SLAB
NSTG = 6


def _bc(x):
    return jnp.broadcast_to(x, (16,))


def _sc_body(numT, catT, w_hbm, bias_hbm, cls_hbm, tabT, out,
             craw, nraw, stg, w_v, bias_v, cls_v,
             gs0, gs1, gs2, gs3, wsem):
    gsems = [gs0, gs1, gs2, gs3]
    wid = lax.axis_index("s") * NC + lax.axis_index("c")
    base = pl.multiple_of(wid * BW, BW)

    pltpu.sync_copy(w_hbm, w_v)
    pltpu.sync_copy(bias_hbm, bias_v)
    pltpu.sync_copy(cls_hbm, cls_v)

    def slab(s, carry):
        b0 = pl.multiple_of(base + s * SLAB, SLAB)
        pltpu.sync_copy(catT.at[:, pl.ds(b0, SLAB)], craw)
        pltpu.sync_copy(numT.at[:, pl.ds(b0, SLAB)], nraw)

        wh = [None] * NSTG

        def prefire(i):
            slot = 2 + (i % 4)
            if wh[slot] is not None:
                wh[slot].wait()
                wh[slot] = None
            idx = craw.at[i]
            sem = gsems[i % 4]

            def dfire(d, c):
                pltpu.async_copy(tabT.at[i, d].at[idx], stg.at[slot, d], sem)
                return c
            lax.fori_loop(0, D, dfire, 0)

        def emit(t, slot):
            wh[slot] = pltpu.async_copy(
                stg.at[slot], out.at[pl.ds(t * D, D), pl.ds(b0, SLAB)], wsem)

        for i in range(3):
            prefire(i)

        def fill_cls(st):
            def dloop(d, c):
                cv = plsc.load_gather(cls_v, [_bc(d)])
                for k in range(SLAB // 16):
                    st[d, pl.ds(16 * k, 16)] = cv
                return c
            lax.fori_loop(0, D, dloop, 0)
        fill_cls(stg.at[0])
        emit(0, 0)

        for j in range(NN):
            slot = (j + 1) % 2
            if wh[slot] is not None:
                wh[slot].wait()

            def fill_num(st, j=j):
                def dloop(d, c):
                    bw = plsc.load_gather(w_v, [_bc(j), _bc(d)])
                    bb = plsc.load_gather(bias_v, [_bc(j), _bc(d)])
                    for k in range(SLAB // 16):
                        st[d, pl.ds(16 * k, 16)] = nraw[j, pl.ds(16 * k, 16)] * bw + bb
                    return c
                lax.fori_loop(0, D, dloop, 0)
            fill_num(stg.at[slot])
            emit(1 + j, slot)

        for i in range(NCAT):
            slot = 2 + (i % 4)
            pltpu.make_async_copy(
                tabT.at[0].at[:, pl.ds(0, SLAB)], stg.at[slot],
                gsems[i % 4]).wait()
            emit(1 + NN + i, slot)
            if i + 3 < NCAT:
                prefire(i + 3)

        for h in wh:
            if h is not None:
                h.wait()
        return carry
    lax.fori_loop(0, NSLAB, slab, 0)


def kernel(numeric, categorical, num_weight, num_bias, cat_tables, cls_token):
    numT = numeric.T
    catT = categorical.T
    tabT = cat_tables.transpose(0, 2, 1)
    cls = cls_token.reshape(D)
    mesh = plsc.VectorSubcoreMesh(core_axis_name="c", subcore_axis_name="s")
    fn = pl.kernel(
        _sc_body,
        out_type=jax.ShapeDtypeStruct((NTOK * D, B), jnp.float32),
        mesh=mesh,
        scratch_types=[
            pltpu.VMEM((NCAT, SLAB), jnp.int32),
            pltpu.VMEM((NN, SLAB), jnp.float32),
            pltpu.VMEM((NSTG, D, SLAB), jnp.float32),
            pltpu.VMEM((NN, D), jnp.float32),
            pltpu.VMEM((NN, D), jnp.float32),
            pltpu.VMEM((D,), jnp.float32),
            pltpu.SemaphoreType.DMA,
            pltpu.SemaphoreType.DMA,
            pltpu.SemaphoreType.DMA,
            pltpu.SemaphoreType.DMA,
            pltpu.SemaphoreType.DMA,
        ],
        compiler_params=pltpu.CompilerParams(use_tc_tiling_on_sc=False,
                                             needs_layout_passes=False),
    )
    out = fn(numT, catT, num_weight, num_bias, cls, tabT)
    return out.reshape(NTOK, D, B).transpose(2, 0, 1)
